# baseline (speedup 1.0000x reference)
import dataclasses
import functools

import jax
import jax.numpy as jnp
from jax import lax
from jax.experimental import pallas as pl
from jax.experimental.pallas import tpu as pltpu
from jax.experimental.pallas import tpu_sc as plsc

N_NODES = 10000
N_EDGES = 320000
D_IN = 128

NC = 2
NS = 16
NW = NC * NS
EDGES_PER_TILE = N_EDGES // NW
CHUNK = 96
NCHUNKS = 108
CHUNK3 = 80
NCHUNKS3 = 129
N_PAD = 10112
ROWS_PER_TILE = N_PAD // NS


@functools.cache
def _mesh():
    return plsc.VectorSubcoreMesh(core_axis_name="c", subcore_axis_name="s",
                                  num_cores=NC, num_subcores=NS)


@functools.cache
def _sc_params():
    cp = pltpu.CompilerParams()
    if "needs_layout_passes" in pltpu.CompilerParams.__dataclass_fields__:
        cp = dataclasses.replace(cp, needs_layout_passes=False)
    return cp


def _sc_aggregate(x, src3, dst3, zeros, with_deg):
    d = x.shape[1]
    nbuf = 2 if with_deg else 3
    CHUNK = src3.shape[2]
    NCHUNKS = src3.shape[1]
    out_type = [jax.ShapeDtypeStruct((NC, N_PAD, d), jnp.float32)]
    scratch = [pltpu.VMEM((NCHUNKS, CHUNK), jnp.int32)]
    scratch += [pltpu.VMEM((CHUNK,), jnp.int32) for _ in range(nbuf)]
    scratch += [pltpu.VMEM((CHUNK, d), jnp.float32) for _ in range(nbuf)]
    scratch += [pltpu.VMEM_SHARED((N_PAD, d), jnp.float32)]
    scratch += [pltpu.SemaphoreType.DMA for _ in range(2 * nbuf)]
    if with_deg:
        out_type.append(jax.ShapeDtypeStruct((NW, N_PAD), jnp.float32))
        scratch.append(pltpu.VMEM((N_PAD,), jnp.float32))

    @functools.partial(
        pl.kernel,
        out_type=tuple(out_type),
        mesh=_mesh(),
        compiler_params=_sc_params(),
        scratch_types=scratch,
    )
    def body(x_hbm, src_hbm, dst_hbm, zeros_hbm, *rest):
        if with_deg:
            (agg_hbm, deg_hbm, dsti, srcb0, srcb1, rows0, rows1, acc,
             sem0, sem1, isem0, isem1, hist) = rest
            srcb = (srcb0, srcb1)
            rows = (rows0, rows1)
            gsem = (sem0, sem1)
            isem = (isem0, isem1)
        else:
            (agg_hbm, dsti, srcb0, srcb1, srcb2, rows0, rows1, rows2, acc,
             sem0, sem1, sem2, isem0, isem1, isem2) = rest
            srcb = (srcb0, srcb1, srcb2)
            rows = (rows0, rows1, rows2)
            gsem = (sem0, sem1, sem2)
            isem = (isem0, isem1, isem2)
        c = lax.axis_index("c")
        s = lax.axis_index("s")
        t = c * NS + s
        base = s * ROWS_PER_TILE

        zero16 = jnp.zeros((16,), jnp.float32)
        one16 = jnp.full((16,), 1.0, jnp.float32)

        icp = pltpu.async_copy(dst_hbm.at[t], dsti, sem0)
        pltpu.sync_copy(zeros_hbm.at[pl.ds(base, ROWS_PER_TILE)],
                        acc.at[pl.ds(base, ROWS_PER_TILE)])

        if with_deg:
            @pl.loop(0, N_PAD // 16)
            def _(r):
                hist[pl.ds(r * 16, 16)] = zero16

        icp.wait()
        plsc.subcore_barrier()

        def hist_update(j):
            @pl.loop(0, CHUNK // 16)
            def _(i):
                idx16 = dsti[j, pl.ds(i * 16, 16)]
                plsc.addupdate_scatter(hist, [idx16], one16)

        assert NCHUNKS % nbuf == 0
        for k in range(nbuf):
            pltpu.sync_copy(src_hbm.at[t, k], srcb[k])
            pltpu.async_copy(x_hbm.at[srcb[k]], rows[k], gsem[k])

        @pl.loop(0, NCHUNKS // nbuf)
        def _(p):
            j0 = nbuf * p
            for k in range(nbuf):
                j = j0 + k
                pltpu.make_async_copy(x_hbm.at[srcb[k]], rows[k],
                                      gsem[k]).wait()

                @pl.when(j + nbuf < NCHUNKS)
                def _(k=k, j=j):
                    pltpu.async_copy(src_hbm.at[t, j + nbuf], srcb[k], isem[k])

                pltpu.sync_copy(rows[k], acc.at[dsti.at[j]], add=True)

                @pl.when(j + nbuf < NCHUNKS)
                def _(k=k, j=j):
                    pltpu.make_async_copy(src_hbm.at[t, j + nbuf], srcb[k],
                                          isem[k]).wait()
                    pltpu.async_copy(x_hbm.at[srcb[k]], rows[k], gsem[k])

                if with_deg:
                    hist_update(j)

        plsc.subcore_barrier()

        pltpu.sync_copy(acc.at[pl.ds(base, ROWS_PER_TILE)],
                        agg_hbm.at[c, pl.ds(base, ROWS_PER_TILE)])
        if with_deg:
            pltpu.sync_copy(hist, deg_hbm.at[t])

    return body(x, src3, dst3, zeros)


def _dense_self(x, w_self, b):
    n, d_in = x.shape
    d_out = w_self.shape[1]
    blk = 2000

    def body(x_ref, ws_ref, b_ref, o_ref):
        o_ref[...] = jnp.dot(x_ref[...], ws_ref[...],
                             preferred_element_type=jnp.float32) + b_ref[...]

    return pl.pallas_call(
        body,
        grid=(n // blk,),
        in_specs=[
            pl.BlockSpec((blk, d_in), lambda i: (i, 0)),
            pl.BlockSpec((d_in, d_out), lambda i: (0, 0)),
            pl.BlockSpec((1, d_out), lambda i: (0, 0)),
        ],
        out_specs=pl.BlockSpec((blk, d_out), lambda i: (i, 0)),
        out_shape=jax.ShapeDtypeStruct((n, d_out), jnp.float32),
    )(x, w_self, b.reshape(1, -1))


def _dense_combine(selfpart, agg2, invdeg, w_neigh, relu):
    n, d_out = selfpart.shape
    d_in = w_neigh.shape[0]
    blk = 2000

    def body(s_ref, a_ref, i_ref, wn_ref, o_ref):
        hn = (a_ref[0] + a_ref[1]) * i_ref[...]
        r = s_ref[...] + jnp.dot(hn, wn_ref[...],
                                 preferred_element_type=jnp.float32)
        if relu:
            r = jnp.maximum(r, 0.0)
        o_ref[...] = r

    return pl.pallas_call(
        body,
        grid=(n // blk,),
        in_specs=[
            pl.BlockSpec((blk, d_out), lambda i: (i, 0)),
            pl.BlockSpec((NC, blk, d_in), lambda i: (0, i, 0)),
            pl.BlockSpec((blk, 1), lambda i: (i, 0)),
            pl.BlockSpec((d_in, d_out), lambda i: (0, 0)),
        ],
        out_specs=pl.BlockSpec((blk, d_out), lambda i: (i, 0)),
        out_shape=jax.ShapeDtypeStruct((n, d_out), jnp.float32),
    )(selfpart, agg2, invdeg, w_neigh)


def _edge_layout(src2, dst2, nchunks, chunk):
    pad = nchunks * chunk - EDGES_PER_TILE
    pad_dst = (N_NODES
               + (jnp.arange(NW * pad, dtype=jnp.int32) % (N_PAD - N_NODES))
               ).reshape(NW, pad)
    pad_src = (jnp.arange(NW * pad, dtype=jnp.int32) * 61 % N_NODES
               ).reshape(NW, pad)
    src3 = jnp.concatenate([src2, pad_src], axis=1).reshape(NW, nchunks, chunk)
    dst3 = jnp.concatenate([dst2, pad_dst], axis=1).reshape(NW, nchunks, chunk)
    return src3, dst3


def kernel(features, edge_index, W_self0, W_neigh0, b0,
           W_self1, W_neigh1, b1, W_self2, W_neigh2, b2):
    src2 = edge_index[0].astype(jnp.int32).reshape(NW, EDGES_PER_TILE)
    dst2 = edge_index[1].astype(jnp.int32).reshape(NW, EDGES_PER_TILE)
    src3a, dst3a = _edge_layout(src2, dst2, NCHUNKS, CHUNK)
    src3b, dst3b = _edge_layout(src2, dst2, NCHUNKS3, CHUNK3)
    zeros = jnp.zeros((N_PAD, D_IN), jnp.float32)

    agg0, hist = _sc_aggregate(features, src3a, dst3a, zeros, with_deg=True)
    self0 = _dense_self(features, W_self0, b0)
    invdeg = (1.0 / jnp.maximum(jnp.sum(hist, axis=0)[:N_NODES], 1.0))[:, None]
    h1 = _dense_combine(self0, agg0, invdeg, W_neigh0, relu=True)
    (agg1,) = _sc_aggregate(h1, src3b, dst3b, zeros, with_deg=False)
    self1 = _dense_self(h1, W_self1, b1)
    h2 = _dense_combine(self1, agg1, invdeg, W_neigh1, relu=True)
    (agg2,) = _sc_aggregate(h2, src3b, dst3b, zeros, with_deg=False)
    self2 = _dense_self(h2, W_self2, b2)
    return _dense_combine(self2, agg2, invdeg, W_neigh2, relu=False)


# speedup vs baseline: 1.0036x; 1.0036x over previous
import dataclasses
import functools

import jax
import jax.numpy as jnp
from jax import lax
from jax.experimental import pallas as pl
from jax.experimental.pallas import tpu as pltpu
from jax.experimental.pallas import tpu_sc as plsc

N_NODES = 10000
N_EDGES = 320000
D_IN = 128

NC = 2
NS = 16
NW = NC * NS
EDGES_PER_TILE = N_EDGES // NW
CHUNK = 96
NCHUNKS = 108
CHUNK3 = 80
NCHUNKS3 = 129
N_PAD = 10112
ROWS_PER_TILE = N_PAD // NS


@functools.cache
def _mesh():
    return plsc.VectorSubcoreMesh(core_axis_name="c", subcore_axis_name="s",
                                  num_cores=NC, num_subcores=NS)


@functools.cache
def _sc_params():
    cp = pltpu.CompilerParams()
    if "needs_layout_passes" in pltpu.CompilerParams.__dataclass_fields__:
        cp = dataclasses.replace(cp, needs_layout_passes=False)
    return cp


def _sc_aggregate(x, src3, dst3, zeros, with_deg):
    d = x.shape[1]
    nbuf = 2 if with_deg else 3
    CHUNK = src3.shape[2]
    NCHUNKS = src3.shape[1]
    out_type = [jax.ShapeDtypeStruct((NC, N_PAD, d), jnp.float32)]
    scratch = [pltpu.VMEM((NCHUNKS, CHUNK), jnp.int32)]
    scratch += [pltpu.VMEM((CHUNK,), jnp.int32) for _ in range(nbuf)]
    scratch += [pltpu.VMEM((CHUNK, d), jnp.float32) for _ in range(nbuf)]
    scratch += [pltpu.VMEM_SHARED((N_PAD, d), jnp.float32)]
    scratch += [pltpu.SemaphoreType.DMA for _ in range(2 * nbuf)]
    if with_deg:
        out_type.append(jax.ShapeDtypeStruct((NW, N_PAD), jnp.float32))
        scratch.append(pltpu.VMEM((N_PAD,), jnp.float32))

    @functools.partial(
        pl.kernel,
        out_type=tuple(out_type),
        mesh=_mesh(),
        compiler_params=_sc_params(),
        scratch_types=scratch,
    )
    def body(x_hbm, src_hbm, dst_hbm, zeros_hbm, *rest):
        if with_deg:
            (agg_hbm, deg_hbm, dsti, srcb0, srcb1, rows0, rows1, acc,
             sem0, sem1, isem0, isem1, hist) = rest
            srcb = (srcb0, srcb1)
            rows = (rows0, rows1)
            gsem = (sem0, sem1)
            isem = (isem0, isem1)
        else:
            (agg_hbm, dsti, srcb0, srcb1, srcb2, rows0, rows1, rows2, acc,
             sem0, sem1, sem2, isem0, isem1, isem2) = rest
            srcb = (srcb0, srcb1, srcb2)
            rows = (rows0, rows1, rows2)
            gsem = (sem0, sem1, sem2)
            isem = (isem0, isem1, isem2)
        c = lax.axis_index("c")
        s = lax.axis_index("s")
        t = c * NS + s
        base = s * ROWS_PER_TILE

        zero16 = jnp.zeros((16,), jnp.float32)
        one16 = jnp.full((16,), 1.0, jnp.float32)

        icp = pltpu.async_copy(dst_hbm.at[t], dsti, sem0)
        pltpu.sync_copy(zeros_hbm.at[pl.ds(base, ROWS_PER_TILE)],
                        acc.at[pl.ds(base, ROWS_PER_TILE)])

        if with_deg:
            @pl.loop(0, N_PAD // 16)
            def _(r):
                hist[pl.ds(r * 16, 16)] = zero16

        icp.wait()
        plsc.subcore_barrier()

        def hist_update(j):
            @pl.loop(0, CHUNK // 16)
            def _(i):
                idx16 = dsti[j, pl.ds(i * 16, 16)]
                plsc.addupdate_scatter(hist, [idx16], one16)

        assert NCHUNKS % nbuf == 0
        for k in range(nbuf):
            pltpu.sync_copy(src_hbm.at[t, k], srcb[k])
            pltpu.async_copy(x_hbm.at[srcb[k]], rows[k], gsem[k])

        @pl.loop(0, NCHUNKS // nbuf)
        def _(p):
            j0 = nbuf * p
            for k in range(nbuf):
                j = j0 + k
                pltpu.make_async_copy(x_hbm.at[srcb[k]], rows[k],
                                      gsem[k]).wait()

                @pl.when(j + nbuf < NCHUNKS)
                def _(k=k, j=j):
                    pltpu.async_copy(src_hbm.at[t, j + nbuf], srcb[k], isem[k])

                pltpu.sync_copy(rows[k], acc.at[dsti.at[j]], add=True)

                @pl.when(j + nbuf < NCHUNKS)
                def _(k=k, j=j):
                    pltpu.make_async_copy(src_hbm.at[t, j + nbuf], srcb[k],
                                          isem[k]).wait()
                    pltpu.async_copy(x_hbm.at[srcb[k]], rows[k], gsem[k])

                if with_deg:
                    hist_update(j)

        plsc.subcore_barrier()

        pltpu.sync_copy(acc.at[pl.ds(base, ROWS_PER_TILE)],
                        agg_hbm.at[c, pl.ds(base, ROWS_PER_TILE)])
        if with_deg:
            pltpu.sync_copy(hist, deg_hbm.at[t])

    return body(x, src3, dst3, zeros)


def _dense(x, agg2, invdeg, w_self, w_neigh, b, relu):
    n, d_in = x.shape
    d_out = w_self.shape[1]
    blk = 2000

    def body(x_ref, a_ref, i_ref, ws_ref, wn_ref, b_ref, o_ref):
        hn = (a_ref[0] + a_ref[1]) * i_ref[...]
        r = jnp.dot(x_ref[...], ws_ref[...], preferred_element_type=jnp.float32)
        r = r + jnp.dot(hn, wn_ref[...], preferred_element_type=jnp.float32)
        r = r + b_ref[...]
        if relu:
            r = jnp.maximum(r, 0.0)
        o_ref[...] = r

    return pl.pallas_call(
        body,
        grid=(n // blk,),
        in_specs=[
            pl.BlockSpec((blk, d_in), lambda i: (i, 0)),
            pl.BlockSpec((NC, blk, d_in), lambda i: (0, i, 0)),
            pl.BlockSpec((blk, 1), lambda i: (i, 0)),
            pl.BlockSpec((d_in, d_out), lambda i: (0, 0)),
            pl.BlockSpec((d_in, d_out), lambda i: (0, 0)),
            pl.BlockSpec((1, d_out), lambda i: (0, 0)),
        ],
        out_specs=pl.BlockSpec((blk, d_out), lambda i: (i, 0)),
        out_shape=jax.ShapeDtypeStruct((n, d_out), jnp.float32),
    )(x, agg2, invdeg, w_self, w_neigh, b.reshape(1, -1))


def _edge_layout(src2, dst2, nchunks, chunk):
    pad = nchunks * chunk - EDGES_PER_TILE
    pad_dst = (N_NODES
               + (jnp.arange(NW * pad, dtype=jnp.int32) % (N_PAD - N_NODES))
               ).reshape(NW, pad)
    pad_src = (jnp.arange(NW * pad, dtype=jnp.int32) * 61 % N_NODES
               ).reshape(NW, pad)
    src3 = jnp.concatenate([src2, pad_src], axis=1).reshape(NW, nchunks, chunk)
    dst3 = jnp.concatenate([dst2, pad_dst], axis=1).reshape(NW, nchunks, chunk)
    return src3, dst3


def kernel(features, edge_index, W_self0, W_neigh0, b0,
           W_self1, W_neigh1, b1, W_self2, W_neigh2, b2):
    src2 = edge_index[0].astype(jnp.int32).reshape(NW, EDGES_PER_TILE)
    dst2 = edge_index[1].astype(jnp.int32).reshape(NW, EDGES_PER_TILE)
    src3a, dst3a = _edge_layout(src2, dst2, NCHUNKS, CHUNK)
    src3b, dst3b = _edge_layout(src2, dst2, NCHUNKS3, CHUNK3)
    zeros = jnp.zeros((N_PAD, D_IN), jnp.float32)

    agg0, hist = _sc_aggregate(features, src3a, dst3a, zeros, with_deg=True)
    invdeg = (1.0 / jnp.maximum(jnp.sum(hist, axis=0)[:N_NODES], 1.0))[:, None]
    h1 = _dense(features, agg0, invdeg, W_self0, W_neigh0, b0, relu=True)
    (agg1,) = _sc_aggregate(h1, src3b, dst3b, zeros, with_deg=False)
    h2 = _dense(h1, agg1, invdeg, W_self1, W_neigh1, b1, relu=True)
    (agg2,) = _sc_aggregate(h2, src3b, dst3b, zeros, with_deg=False)
    return _dense(h2, agg2, invdeg, W_self2, W_neigh2, b2, relu=False)
